# baseline (speedup 1.0000x reference)
import functools

import jax
import jax.numpy as jnp
from jax import lax
from jax.experimental import pallas as pl
from jax.experimental.pallas import tpu as pltpu
from jax.experimental.pallas import tpu_sc as plsc

_NC = 2
_NS = 16
_NW = _NC * _NS
_LANES = 16
_CHUNK = 10000


@functools.partial(jax.jit, static_argnames=("e", "n_nodes"))
def _degree_partials(adjs_flat, e, n_nodes):
    epw = e // _NW
    nchunk = epw // _CHUNK
    assert epw % _CHUNK == 0 and e % _NW == 0 and n_nodes % _LANES == 0

    mesh = plsc.VectorSubcoreMesh(
        core_axis_name="c", subcore_axis_name="s",
        num_cores=_NC, num_subcores=_NS)

    @functools.partial(
        pl.kernel,
        out_type=jax.ShapeDtypeStruct((_NW * n_nodes,), jnp.float32),
        mesh=mesh,
        scratch_types=[
            pltpu.VMEM((_CHUNK,), jnp.int32),
            pltpu.VMEM((_CHUNK,), jnp.int32),
            pltpu.VMEM((n_nodes,), jnp.float32),
            pltpu.SemaphoreType.DMA,
            pltpu.SemaphoreType.DMA,
        ],
        compiler_params=pltpu.CompilerParams(needs_layout_passes=False),
    )
    def hist(adjs_hbm, part_hbm, idx0, idx1, acc, sem0, sem1):
        wid = lax.axis_index("s") * _NC + lax.axis_index("c")
        base = e + wid * epw

        bufs = (idx0, idx1)
        sems = (sem0, sem1)
        copies = [None, None]
        copies[0] = pltpu.async_copy(
            adjs_hbm.at[pl.ds(base, _CHUNK)], idx0, sem0)
        if nchunk > 1:
            copies[1] = pltpu.async_copy(
                adjs_hbm.at[pl.ds(base + _CHUNK, _CHUNK)], idx1, sem1)

        zeros16 = jnp.zeros((_LANES,), jnp.float32)

        @plsc.parallel_loop(0, n_nodes // _LANES, step=1, unroll=32)
        def _(i):
            acc[pl.ds(i * _LANES, _LANES)] = zeros16

        ones16 = jnp.ones((_LANES,), jnp.float32)

        for g in range(nchunk):
            copies[g % 2].wait()
            buf = bufs[g % 2]

            @plsc.parallel_loop(0, _CHUNK // _LANES, step=1, unroll=32)
            def _(i):
                idx = buf[pl.ds(i * _LANES, _LANES)]
                plsc.addupdate_scatter(acc, [idx], ones16)

            if g + 2 < nchunk:
                nb = g % 2
                copies[nb] = pltpu.async_copy(
                    adjs_hbm.at[pl.ds(base + (g + 2) * _CHUNK, _CHUNK)],
                    bufs[nb], sems[nb])

        pltpu.sync_copy(acc, part_hbm.at[pl.ds(wid * n_nodes, n_nodes)])

    return hist(adjs_flat)


def _finalize(x_row, wl_row, bl_row, parts):
    n = x_row.shape[1]

    def body(x_ref, w_ref, b_ref, p_ref, o_ref):
        a = jnp.sum(w_ref[...])
        b = jnp.sum(b_ref[...])
        deg = jnp.sum(p_ref[...], axis=0, keepdims=True)
        o_ref[...] = x_ref[...] + jnp.maximum(deg, 1.0) * a + b

    return pl.pallas_call(
        body,
        out_shape=jax.ShapeDtypeStruct((1, n), jnp.float32),
    )(x_row, wl_row, bl_row, parts)


def kernel(x, adjs, Wl, bl, Wf, bf):
    n = x.shape[0]
    nl = Wl.shape[0]
    parts = _degree_partials(adjs.reshape(-1), adjs.shape[1], n)
    out = _finalize(
        x.reshape(1, n),
        Wl.reshape(1, nl),
        bl.reshape(1, nl),
        parts.reshape(_NW, n),
    )
    return out.reshape(n)


# speedup vs baseline: 1.0543x; 1.0543x over previous
import functools

import jax
import jax.numpy as jnp
from jax import lax
from jax.experimental import pallas as pl
from jax.experimental.pallas import tpu as pltpu
from jax.experimental.pallas import tpu_sc as plsc

_NC = 2
_NS = 16
_NW = _NC * _NS
_LANES = 16
_CHUNK = 10000


@functools.partial(jax.jit, static_argnames=("e", "n_nodes"))
def _degree_partials(adjs_flat, e, n_nodes):
    epw = e // _NW
    nchunk = epw // _CHUNK
    assert epw % _CHUNK == 0 and e % _NW == 0 and n_nodes % _LANES == 0

    mesh = plsc.VectorSubcoreMesh(
        core_axis_name="c", subcore_axis_name="s",
        num_cores=_NC, num_subcores=_NS)

    @functools.partial(
        pl.kernel,
        out_type=jax.ShapeDtypeStruct((_NW * n_nodes,), jnp.float32),
        mesh=mesh,
        scratch_types=[
            pltpu.VMEM((_CHUNK,), jnp.int32),
            pltpu.VMEM((_CHUNK,), jnp.int32),
            pltpu.VMEM((n_nodes,), jnp.float32),
            pltpu.SemaphoreType.DMA,
            pltpu.SemaphoreType.DMA,
        ],
        compiler_params=pltpu.CompilerParams(needs_layout_passes=False),
    )
    def hist(adjs_hbm, part_hbm, idx0, idx1, acc, sem0, sem1):
        wid = lax.axis_index("s") * _NC + lax.axis_index("c")
        base = e + wid * epw

        bufs = (idx0, idx1)
        sems = (sem0, sem1)
        copies = [None, None]
        copies[0] = pltpu.async_copy(
            adjs_hbm.at[pl.ds(base, _CHUNK)], idx0, sem0)
        if nchunk > 1:
            copies[1] = pltpu.async_copy(
                adjs_hbm.at[pl.ds(base + _CHUNK, _CHUNK)], idx1, sem1)

        zeros16 = jnp.zeros((_LANES,), jnp.float32)

        @plsc.parallel_loop(0, n_nodes // _LANES, step=1, unroll=8)
        def _(i):
            acc[pl.ds(i * _LANES, _LANES)] = zeros16

        ones16 = jnp.ones((_LANES,), jnp.float32)

        for g in range(nchunk):
            copies[g % 2].wait()
            buf = bufs[g % 2]

            @plsc.parallel_loop(0, _CHUNK // _LANES, step=1, unroll=8)
            def _(i):
                idx = buf[pl.ds(i * _LANES, _LANES)]
                plsc.addupdate_scatter(acc, [idx], ones16)

            if g + 2 < nchunk:
                nb = g % 2
                copies[nb] = pltpu.async_copy(
                    adjs_hbm.at[pl.ds(base + (g + 2) * _CHUNK, _CHUNK)],
                    bufs[nb], sems[nb])

        pltpu.sync_copy(acc, part_hbm.at[pl.ds(wid * n_nodes, n_nodes)])

    return hist(adjs_flat)


def _finalize(x_row, wl_row, bl_row, parts):
    n = x_row.shape[1]

    def body(x_ref, w_ref, b_ref, p_ref, o_ref):
        a = jnp.sum(w_ref[...])
        b = jnp.sum(b_ref[...])
        deg = jnp.sum(p_ref[...], axis=0, keepdims=True)
        o_ref[...] = x_ref[...] + jnp.maximum(deg, 1.0) * a + b

    return pl.pallas_call(
        body,
        out_shape=jax.ShapeDtypeStruct((1, n), jnp.float32),
    )(x_row, wl_row, bl_row, parts)


def kernel(x, adjs, Wl, bl, Wf, bf):
    n = x.shape[0]
    nl = Wl.shape[0]
    parts = _degree_partials(adjs.reshape(-1), adjs.shape[1], n)
    out = _finalize(
        x.reshape(1, n),
        Wl.reshape(1, nl),
        bl.reshape(1, nl),
        parts.reshape(_NW, n),
    )
    return out.reshape(n)
